# Initial kernel scaffold; baseline (speedup 1.0000x reference)
import jax
import jax.numpy as jnp
from jax.experimental import pallas as pl


def kernel(feats, coords, cu_seqlens, pos2d_w, pos_t_w, pos_z_w):
    raise NotImplementedError("write your pallas kernel here")


# speedup vs baseline: 36.9357x; 36.9357x over previous
import numpy as np
import jax
import jax.numpy as jnp
from jax.experimental import pallas as pl

_GRID = 16
_KDIM = 384
_EPS1000 = np.float32(1000.0 * np.finfo(np.float32).eps)


def _embed_kernel(xs_ref, ys_ref, ts_ref, zs_ref, feats_ref, table_ref, out_ref):
    xs = xs_ref[...]
    ys = ys_ref[...]
    ts = ts_ref[...]
    zs = zs_ref[...]
    s = xs.shape[0]

    lane = jax.lax.broadcasted_iota(jnp.int32, (1, _KDIM), 1)
    spatial = lane < _GRID * _GRID

    def axis_weights(coord, idx_lane):
        out_size = jnp.max(coord, keepdims=True).astype(jnp.float32) + 1.0
        inv_scale = jnp.float32(_GRID) / out_size
        kscale = jnp.maximum(inv_scale, 1.0)
        sf = (coord.astype(jnp.float32) + 0.5) * inv_scale - 0.5
        d = jnp.abs(sf - idx_lane.astype(jnp.float32)) / kscale
        w = jnp.maximum(0.0, 1.0 - d)
        w = jnp.where(spatial, w, 0.0)
        tot = jnp.sum(w, axis=1, keepdims=True) * jnp.float32(1.0 / _GRID)
        safe = jnp.where(tot != 0.0, tot, 1.0)
        fac = jnp.where(jnp.abs(tot) > _EPS1000, 1.0 / safe, 0.0)
        inb = jnp.logical_and(sf >= -0.5, sf <= jnp.float32(_GRID) - 0.5)
        fac = jnp.where(inb, fac, 0.0)
        return w, fac

    wx, fx = axis_weights(xs, lane // _GRID)
    wy, fy = axis_weights(ys, lane % _GRID)
    w_sp = wx * wy * (fx * fy)

    onehot = jnp.logical_or(lane - 256 == ts, lane - 288 == zs)
    w = jnp.where(spatial, w_sp, onehot.astype(jnp.float32))

    acc = jax.lax.dot_general(
        w, table_ref[...], (((1,), (0,)), ((), ())),
        preferred_element_type=jnp.float32)
    out_ref[...] = feats_ref[...] + acc


def kernel(feats, coords, cu_seqlens, pos2d_w, pos_t_w, pos_z_w):
    tot, hid = feats.shape
    nb = cu_seqlens.shape[0] - 1
    seg = tot // nb
    pad = _KDIM - (pos2d_w.shape[0] + pos_t_w.shape[0] + pos_z_w.shape[0])
    table = jnp.concatenate(
        [pos2d_w, pos_t_w, pos_z_w, jnp.zeros((pad, hid), jnp.float32)], axis=0)
    ts = coords[:, 1:2]
    xs = coords[:, 2:3]
    ys = coords[:, 3:4]
    zs = coords[:, 4:5]

    col = pl.BlockSpec((seg, 1), lambda b: (b, 0))
    return pl.pallas_call(
        _embed_kernel,
        grid=(nb,),
        in_specs=[
            col, col, col, col,
            pl.BlockSpec((seg, hid), lambda b: (b, 0)),
            pl.BlockSpec((_KDIM, hid), lambda b: (0, 0)),
        ],
        out_specs=pl.BlockSpec((seg, hid), lambda b: (b, 0)),
        out_shape=jax.ShapeDtypeStruct((tot, hid), jnp.float32),
    )(xs, ys, ts, zs, feats, table)
